# baseline (speedup 1.0000x reference)
import functools

import jax
import jax.numpy as jnp
from jax import lax
from jax.experimental import pallas as pl
from jax.experimental.pallas import tpu as pltpu
from jax.experimental.pallas import tpu_sc as plsc

VOCAB = 1000000
DIM = 64
B = 16384

_INFO = plsc.get_sparse_core_info()
NC = _INFO.num_cores
NS = _INFO.num_subcores
L = _INFO.num_lanes
NW = NC * NS
BPW = B // NW
CHUNK = 128
NCHUNK = BPW // CHUNK
NGROUP = BPW // L


def _glove_body(center_hbm, target_hbm, coocs_hbm, weights_hbm,
                embv_hbm, embu_hbm, vb_hbm, ub_hbm, out_hbm,
                cidx, tidx, vrows, urows, cb, tb, cc, ww, accv, sem):
    wid = lax.axis_index("s") * NC + lax.axis_index("c")

    pltpu.sync_copy(center_hbm.at[wid], cidx)
    pltpu.sync_copy(target_hbm.at[wid], tidx)
    pltpu.sync_copy(coocs_hbm.at[wid], cc)
    pltpu.sync_copy(weights_hbm.at[wid], ww)

    for j in range(NCHUNK):
        rs = pl.ds(j * CHUNK, CHUNK)
        pltpu.async_copy(embv_hbm.at[cidx.at[j]], vrows.at[rs], sem).wait()
        pltpu.async_copy(embu_hbm.at[tidx.at[j]], urows.at[rs], sem).wait()
        pltpu.async_copy(vb_hbm.at[cidx.at[j]], cb.at[rs], sem).wait()
        pltpu.async_copy(ub_hbm.at[tidx.at[j]], tb.at[rs], sem).wait()

    def body(g, acc):
        rows = g * L + lax.iota(jnp.int32, L)
        dot = jnp.zeros((L,), jnp.float32)
        for d in range(DIM):
            col = jnp.full((L,), d, jnp.int32)
            vv = plsc.load_gather(vrows, [rows, col])
            uu = plsc.load_gather(urows, [rows, col])
            dot = dot + vv * uu
        s = pl.ds(g * L, L)
        r = dot + cb[s] + tb[s] - cc[s]
        return acc + ww[s] * r * r

    total = lax.fori_loop(0, NGROUP, body, jnp.zeros((L,), jnp.float32))
    accv[...] = total
    pltpu.sync_copy(accv, out_hbm.at[wid])


@jax.jit
def _glove_sc(center, target, coocs, weights, emb_v, emb_u, vb, ub):
    mesh = plsc.VectorSubcoreMesh(core_axis_name="c", subcore_axis_name="s")
    fn = pl.kernel(
        _glove_body,
        mesh=mesh,
        compiler_params=pltpu.CompilerParams(
            needs_layout_passes=False, use_tc_tiling_on_sc=False),
        out_type=jax.ShapeDtypeStruct((NW, L), jnp.float32),
        scratch_types=[
            pltpu.VMEM((NCHUNK, CHUNK), jnp.int32),
            pltpu.VMEM((NCHUNK, CHUNK), jnp.int32),
            pltpu.VMEM((BPW, DIM), jnp.float32),
            pltpu.VMEM((BPW, DIM), jnp.float32),
            pltpu.VMEM((BPW,), jnp.float32),
            pltpu.VMEM((BPW,), jnp.float32),
            pltpu.VMEM((BPW,), jnp.float32),
            pltpu.VMEM((BPW,), jnp.float32),
            pltpu.VMEM((L,), jnp.float32),
            pltpu.SemaphoreType.DMA,
        ],
    )
    return fn(center, target, coocs, weights, emb_v, emb_u, vb, ub)


def kernel(center_words, target_words, coocs, weights, emb_v, emb_u, v_bias, u_bias):
    center = center_words.reshape(NW, NCHUNK, CHUNK)
    target = target_words.reshape(NW, NCHUNK, CHUNK)
    cc = coocs.reshape(NW, BPW)
    ww = weights.reshape(NW, BPW)
    partials = _glove_sc(center, target, cc, ww, emb_v, emb_u,
                         v_bias.reshape(VOCAB), u_bias.reshape(VOCAB))
    return jnp.sum(partials)
